# baseline (speedup 1.0000x reference)
import functools

import jax
import jax.numpy as jnp
from jax.experimental import pallas as pl
from jax.experimental.pallas import tpu as pltpu


def _lstm_body(x_ref, wih0_ref, whh0_ref, b0_ref, wih1_ref, whh1_ref, b1_ref,
               wfc_ref, bfc_ref, out_ref, gx_ref, seq_ref, *, T, Bb, H):
    bf = jnp.bfloat16
    col = jax.lax.broadcasted_iota(jnp.int32, (1, 4 * H), 1)
    scl = jnp.where((col >= 2 * H) & (col < 3 * H), 1.0, 0.5)

    wih0 = (wih0_ref[...] * scl).astype(bf)
    b0 = b0_ref[...] * scl

    for t in range(T):
        xt = x_ref[:, t, :].astype(bf)
        gx_ref[t] = (
            jnp.dot(xt, wih0, preferred_element_type=jnp.float32) + b0
        )

    def cell(pre, c):
        tt = jnp.tanh(pre)
        i_g = 0.5 + 0.5 * tt[:, 0 * H:1 * H]
        f_g = 0.5 + 0.5 * tt[:, 1 * H:2 * H]
        g_g = tt[:, 2 * H:3 * H]
        o_g = 0.5 + 0.5 * tt[:, 3 * H:4 * H]
        c = f_g * c + i_g * g_g
        h = o_g * jnp.tanh(c)
        return h, c

    whh0 = (whh0_ref[...] * scl).astype(bf)
    h = jnp.zeros((Bb, H), jnp.float32)
    c = h
    for t in range(T):
        pre = gx_ref[t] + jnp.dot(
            h.astype(bf), whh0, preferred_element_type=jnp.float32
        )
        h, c = cell(pre, c)
        seq_ref[t] = h.astype(bf)

    wih1 = (wih1_ref[...] * scl).astype(bf)
    gx_ref[...] = (
        jnp.dot(seq_ref[...].reshape(T * Bb, H), wih1,
                preferred_element_type=jnp.float32).reshape(T, Bb, 4 * H)
        + b1_ref[...] * scl
    )

    whh1 = (whh1_ref[...] * scl).astype(bf)
    h = jnp.zeros((Bb, H), jnp.float32)
    c = h
    for t in range(T):
        pre = gx_ref[t] + jnp.dot(
            h.astype(bf), whh1, preferred_element_type=jnp.float32
        )
        h, c = cell(pre, c)

    out_ref[...] = (
        jnp.dot(h.astype(bf), wfc_ref[...].astype(bf),
                preferred_element_type=jnp.float32)
        + bfc_ref[...]
    )


@functools.partial(jax.jit, static_argnames=("block_b",))
def _forward(x, w_ih_0, w_hh_0, b_0, w_ih_1, w_hh_1, b_1, w_fc, b_fc,
             block_b=128):
    B, T, I = x.shape
    H = w_hh_0.shape[0]
    C = w_fc.shape[1]
    Bb = min(block_b, ((B + 7) // 8) * 8)
    Bp = ((B + Bb - 1) // Bb) * Bb
    if Bp != B:
        x = jnp.pad(x, ((0, Bp - B), (0, 0), (0, 0)))

    body = functools.partial(_lstm_body, T=T, Bb=Bb, H=H)
    bcast = lambda shape: pl.BlockSpec(shape, lambda i: (0,) * len(shape))
    out = pl.pallas_call(
        body,
        out_shape=jax.ShapeDtypeStruct((Bp, C), jnp.float32),
        grid=(Bp // Bb,),
        in_specs=[
            pl.BlockSpec((Bb, T, I), lambda i: (i, 0, 0)),
            bcast((I, 4 * H)), bcast((H, 4 * H)), bcast((1, 4 * H)),
            bcast((H, 4 * H)), bcast((H, 4 * H)), bcast((1, 4 * H)),
            bcast((H, C)), bcast((1, C)),
        ],
        out_specs=pl.BlockSpec((Bb, C), lambda i: (i, 0)),
        scratch_shapes=[
            pltpu.VMEM((T, Bb, 4 * H), jnp.float32),
            pltpu.VMEM((T, Bb, H), jnp.bfloat16),
        ],
        compiler_params=pltpu.CompilerParams(
            dimension_semantics=("parallel",),
        ),
    )(x, w_ih_0, w_hh_0, b_0, w_ih_1, w_hh_1, b_1, w_fc, b_fc)
    return out[:B]


def kernel(x, w_ih_0, w_hh_0, b_0, w_ih_1, w_hh_1, b_1, w_fc, b_fc):
    return _forward(x, w_ih_0, w_hh_0, b_0, w_ih_1, w_hh_1, b_1, w_fc, b_fc)


# speedup vs baseline: 1.3500x; 1.3500x over previous
import functools

import jax
import jax.numpy as jnp
from jax.experimental import pallas as pl
from jax.experimental.pallas import tpu as pltpu

_CHUNKS = 4


def _lstm_body(x_hbm, wih0_ref, whh0_ref, b0_ref, wih1_ref, whh1_ref, b1_ref,
               wfc_ref, bfc_ref, out_ref, gx_ref, seq_ref, xtm_ref, sem,
               *, T, Bb, H):
    bf = jnp.bfloat16
    i = pl.program_id(0)
    Bc = Bb // _CHUNKS
    col = jax.lax.broadcasted_iota(jnp.int32, (1, 4 * H), 1)
    scl = jnp.where((col >= 2 * H) & (col < 3 * H), 1.0, 0.5)

    def row_copy(b, ch):
        return pltpu.make_async_copy(
            x_hbm.at[i * Bb + b], xtm_ref.at[:, b, :], sem.at[ch]
        )

    for ch in range(_CHUNKS):
        for b in range(ch * Bc, (ch + 1) * Bc):
            row_copy(b, ch).start()

    wih0 = (wih0_ref[...] * scl).astype(bf)
    b0 = b0_ref[...] * scl

    for ch in range(_CHUNKS):
        for b in range(ch * Bc, (ch + 1) * Bc):
            row_copy(b, ch).wait()
        for t in range(T):
            xt = xtm_ref[t, pl.ds(ch * Bc, Bc), :].astype(bf)
            gx_ref[t, pl.ds(ch * Bc, Bc), :] = (
                jnp.dot(xt, wih0, preferred_element_type=jnp.float32) + b0
            )

    def cell(pre, c):
        tt = jnp.tanh(pre)
        i_g = 0.5 + 0.5 * tt[:, 0 * H:1 * H]
        f_g = 0.5 + 0.5 * tt[:, 1 * H:2 * H]
        g_g = tt[:, 2 * H:3 * H]
        o_g = 0.5 + 0.5 * tt[:, 3 * H:4 * H]
        c = f_g * c + i_g * g_g
        h = o_g * jnp.tanh(c)
        return h, c

    whh0 = (whh0_ref[...] * scl).astype(bf)
    h = jnp.zeros((Bb, H), jnp.float32)
    c = h
    for t in range(T):
        pre = gx_ref[t] + jnp.dot(
            h.astype(bf), whh0, preferred_element_type=jnp.float32
        )
        h, c = cell(pre, c)
        seq_ref[t] = h.astype(bf)

    wih1 = (wih1_ref[...] * scl).astype(bf)
    gx_ref[...] = (
        jnp.dot(seq_ref[...].reshape(T * Bb, H), wih1,
                preferred_element_type=jnp.float32).reshape(T, Bb, 4 * H)
        + b1_ref[...] * scl
    )

    whh1 = (whh1_ref[...] * scl).astype(bf)
    h = jnp.zeros((Bb, H), jnp.float32)
    c = h
    for t in range(T):
        pre = gx_ref[t] + jnp.dot(
            h.astype(bf), whh1, preferred_element_type=jnp.float32
        )
        h, c = cell(pre, c)

    out_ref[...] = (
        jnp.dot(h.astype(bf), wfc_ref[...].astype(bf),
                preferred_element_type=jnp.float32)
        + bfc_ref[...]
    )


@functools.partial(jax.jit, static_argnames=("block_b",))
def _forward(x, w_ih_0, w_hh_0, b_0, w_ih_1, w_hh_1, b_1, w_fc, b_fc,
             block_b=256):
    B, T, I = x.shape
    H = w_hh_0.shape[0]
    C = w_fc.shape[1]
    Bb = min(block_b, ((B + 7) // 8) * 8)
    Bp = ((B + Bb - 1) // Bb) * Bb
    if Bp != B:
        x = jnp.pad(x, ((0, Bp - B), (0, 0), (0, 0)))

    body = functools.partial(_lstm_body, T=T, Bb=Bb, H=H)
    bcast = lambda shape: pl.BlockSpec(shape, lambda i: (0,) * len(shape))
    out = pl.pallas_call(
        body,
        out_shape=jax.ShapeDtypeStruct((Bp, C), jnp.float32),
        grid=(Bp // Bb,),
        in_specs=[
            pl.BlockSpec(memory_space=pl.ANY),
            bcast((I, 4 * H)), bcast((H, 4 * H)), bcast((1, 4 * H)),
            bcast((H, 4 * H)), bcast((H, 4 * H)), bcast((1, 4 * H)),
            bcast((H, C)), bcast((1, C)),
        ],
        out_specs=pl.BlockSpec((Bb, C), lambda i: (i, 0)),
        scratch_shapes=[
            pltpu.VMEM((T, Bb, 4 * H), jnp.float32),
            pltpu.VMEM((T, Bb, H), jnp.bfloat16),
            pltpu.VMEM((T, Bb, I), jnp.float32),
            pltpu.SemaphoreType.DMA((_CHUNKS,)),
        ],
        compiler_params=pltpu.CompilerParams(
            dimension_semantics=("parallel",),
        ),
    )(x, w_ih_0, w_hh_0, b_0, w_ih_1, w_hh_1, b_1, w_fc, b_fc)
    return out[:B]


def kernel(x, w_ih_0, w_hh_0, b_0, w_ih_1, w_hh_1, b_1, w_fc, b_fc):
    return _forward(x, w_ih_0, w_hh_0, b_0, w_ih_1, w_hh_1, b_1, w_fc, b_fc)


# speedup vs baseline: 1.6513x; 1.2231x over previous
import functools

import jax
import jax.numpy as jnp
from jax.experimental import pallas as pl
from jax.experimental.pallas import tpu as pltpu

_CHUNKS = 4


def _lstm_body(x_ref, wih0_ref, whh0_ref, b0_ref, wih1_ref, whh1_ref, b1_ref,
               wfc_ref, bfc_ref, out_ref, gx_ref, seq_ref, xtm_ref,
               *, T, Bb, H):
    bf = jnp.bfloat16
    col = jax.lax.broadcasted_iota(jnp.int32, (1, 4 * H), 1)
    scl = jnp.where((col >= 2 * H) & (col < 3 * H), 1.0, 0.5)

    wih0 = (wih0_ref[...] * scl).astype(bf)
    b0 = b0_ref[...] * scl

    xtm_ref[...] = jnp.swapaxes(x_ref[...], 0, 1)
    for t in range(T):
        xt = xtm_ref[t].astype(bf)
        gx_ref[t] = (
            jnp.dot(xt, wih0, preferred_element_type=jnp.float32) + b0
        )

    def cell(pre, c):
        tt = jnp.tanh(pre)
        i_g = 0.5 + 0.5 * tt[:, 0 * H:1 * H]
        f_g = 0.5 + 0.5 * tt[:, 1 * H:2 * H]
        g_g = tt[:, 2 * H:3 * H]
        o_g = 0.5 + 0.5 * tt[:, 3 * H:4 * H]
        c = f_g * c + i_g * g_g
        h = o_g * jnp.tanh(c)
        return h, c

    whh0 = (whh0_ref[...] * scl).astype(bf)
    h = jnp.zeros((Bb, H), jnp.float32)
    c = h
    for t in range(T):
        pre = gx_ref[t] + jnp.dot(
            h.astype(bf), whh0, preferred_element_type=jnp.float32
        )
        h, c = cell(pre, c)
        seq_ref[t] = h.astype(bf)

    wih1 = (wih1_ref[...] * scl).astype(bf)
    gx_ref[...] = (
        jnp.dot(seq_ref[...].reshape(T * Bb, H), wih1,
                preferred_element_type=jnp.float32).reshape(T, Bb, 4 * H)
        + b1_ref[...] * scl
    )

    whh1 = (whh1_ref[...] * scl).astype(bf)
    h = jnp.zeros((Bb, H), jnp.float32)
    c = h
    for t in range(T):
        pre = gx_ref[t] + jnp.dot(
            h.astype(bf), whh1, preferred_element_type=jnp.float32
        )
        h, c = cell(pre, c)

    out_ref[...] = (
        jnp.dot(h.astype(bf), wfc_ref[...].astype(bf),
                preferred_element_type=jnp.float32)
        + bfc_ref[...]
    )


@functools.partial(jax.jit, static_argnames=("block_b",))
def _forward(x, w_ih_0, w_hh_0, b_0, w_ih_1, w_hh_1, b_1, w_fc, b_fc,
             block_b=256):
    B, T, I = x.shape
    H = w_hh_0.shape[0]
    C = w_fc.shape[1]
    Bb = min(block_b, ((B + 7) // 8) * 8)
    Bp = ((B + Bb - 1) // Bb) * Bb
    if Bp != B:
        x = jnp.pad(x, ((0, Bp - B), (0, 0), (0, 0)))

    body = functools.partial(_lstm_body, T=T, Bb=Bb, H=H)
    bcast = lambda shape: pl.BlockSpec(shape, lambda i: (0,) * len(shape))
    out = pl.pallas_call(
        body,
        out_shape=jax.ShapeDtypeStruct((Bp, C), jnp.float32),
        grid=(Bp // Bb,),
        in_specs=[
            pl.BlockSpec((Bb, T, I), lambda i: (i, 0, 0)),
            bcast((I, 4 * H)), bcast((H, 4 * H)), bcast((1, 4 * H)),
            bcast((H, 4 * H)), bcast((H, 4 * H)), bcast((1, 4 * H)),
            bcast((H, C)), bcast((1, C)),
        ],
        out_specs=pl.BlockSpec((Bb, C), lambda i: (i, 0)),
        scratch_shapes=[
            pltpu.VMEM((T, Bb, 4 * H), jnp.float32),
            pltpu.VMEM((T, Bb, H), jnp.bfloat16),
            pltpu.VMEM((T, Bb, I), jnp.float32),
        ],
        compiler_params=pltpu.CompilerParams(
            dimension_semantics=("parallel",),
        ),
    )(x, w_ih_0, w_hh_0, b_0, w_ih_1, w_hh_1, b_1, w_fc, b_fc)
    return out[:B]


def kernel(x, w_ih_0, w_hh_0, b_0, w_ih_1, w_hh_1, b_1, w_fc, b_fc):
    return _forward(x, w_ih_0, w_hh_0, b_0, w_ih_1, w_hh_1, b_1, w_fc, b_fc)
